# baseline (speedup 1.0000x reference)
import jax
import jax.numpy as jnp
from jax import lax
from jax.experimental import pallas as pl

_B = 4096
_CV = 72
_ROW = _CV * 3
_IMG = _CV * _ROW
_BB = 8


def _tc_body(pos_ref, col_ref, out_ref):
    l = lax.broadcasted_iota(jnp.int32, (1, _IMG), 1)
    r = l // _ROW
    c = (l % _ROW) // 3
    ch = l % 3
    t0 = pos_ref[:, 0:1]
    t1 = pos_ref[:, 1:2]
    b0 = pos_ref[:, 2:3]
    b1 = pos_ref[:, 3:4]
    mask = (r >= t0) & (r < b0) & (c >= t1) & (c < b1)
    colv = jnp.where(ch == 0, col_ref[:, 0:1],
                     jnp.where(ch == 1, col_ref[:, 1:2], col_ref[:, 2:3]))
    out_ref[...] = jnp.where(mask, colv, jnp.float32(255.0))


def kernel(positions, colors):
    out2 = pl.pallas_call(
        _tc_body,
        grid=(_B // _BB,),
        in_specs=[pl.BlockSpec((_BB, 4), lambda i: (i, 0)),
                  pl.BlockSpec((_BB, 3), lambda i: (i, 0))],
        out_specs=pl.BlockSpec((_BB, _IMG), lambda i: (i, 0)),
        out_shape=jax.ShapeDtypeStruct((_B, _IMG), jnp.float32),
    )(positions, colors)
    return out2.reshape(_B, _CV, _CV, 3)


# speedup vs baseline: 17.0014x; 17.0014x over previous
import jax
import jax.numpy as jnp
from jax import lax
from jax.experimental import pallas as pl

_B = 4096
_CV = 72
_RB = 8
_BB = 512


def _tc_body(pos_ref, col_ref, out_ref):
    i = pl.program_id(0)
    t0 = pos_ref[0:1]
    t1 = pos_ref[1:2]
    b0 = pos_ref[2:3]
    b1 = pos_ref[3:4]
    riota = lax.broadcasted_iota(jnp.int32, (_RB, 1, 1, _BB), 0) + i * _RB
    ciota = lax.broadcasted_iota(jnp.int32, (1, 1, _CV, _BB), 2)
    rowm = (riota >= t0) & (riota < b0)
    colm = (ciota >= t1) & (ciota < b1)
    mask = rowm & colm
    colv = col_ref[...].reshape(1, 3, 1, _BB)
    out_ref[...] = jnp.where(mask, colv, jnp.float32(255.0))


def kernel(positions, colors):
    posr = positions.T.reshape(4, 1, 1, _B)
    colr = colors.T.reshape(3, 1, 1, _B)
    y = pl.pallas_call(
        _tc_body,
        grid=(_CV // _RB, _B // _BB),
        in_specs=[
            pl.BlockSpec((4, 1, 1, _BB), lambda i, j: (0, 0, 0, j)),
            pl.BlockSpec((3, 1, 1, _BB), lambda i, j: (0, 0, 0, j)),
        ],
        out_specs=pl.BlockSpec((_RB, 3, _CV, _BB), lambda i, j: (i, 0, 0, j)),
        out_shape=jax.ShapeDtypeStruct((_CV, 3, _CV, _B), jnp.float32),
    )(posr, colr)
    return jnp.transpose(y, (3, 0, 2, 1))


# speedup vs baseline: 17.9783x; 1.0575x over previous
import jax
import jax.numpy as jnp
from jax import lax
from jax.experimental import pallas as pl

_B = 4096
_CV = 72
_RB = 8
_BB = 1024


def _tc_body(pos_ref, col_ref, out_ref):
    i = pl.program_id(0)
    t0 = pos_ref[0:1]
    t1 = pos_ref[1:2]
    b0 = pos_ref[2:3]
    b1 = pos_ref[3:4]
    riota = lax.broadcasted_iota(jnp.int32, (_RB, 1, 1, _BB), 0) + i * _RB
    ciota = lax.broadcasted_iota(jnp.int32, (1, 1, _CV, _BB), 2)
    rowm = (riota >= t0) & (riota < b0)
    colm = (ciota >= t1) & (ciota < b1)
    mask = rowm & colm
    colv = col_ref[...].reshape(1, 3, 1, _BB)
    out_ref[...] = jnp.where(mask, colv, jnp.float32(255.0))


def kernel(positions, colors):
    posr = positions.T.reshape(4, 1, 1, _B)
    colr = colors.T.reshape(3, 1, 1, _B)
    y = pl.pallas_call(
        _tc_body,
        grid=(_CV // _RB, _B // _BB),
        in_specs=[
            pl.BlockSpec((4, 1, 1, _BB), lambda i, j: (0, 0, 0, j)),
            pl.BlockSpec((3, 1, 1, _BB), lambda i, j: (0, 0, 0, j)),
        ],
        out_specs=pl.BlockSpec((_RB, 3, _CV, _BB), lambda i, j: (i, 0, 0, j)),
        out_shape=jax.ShapeDtypeStruct((_CV, 3, _CV, _B), jnp.float32),
    )(posr, colr)
    return jnp.transpose(y, (3, 0, 2, 1))
